# Initial kernel scaffold; baseline (speedup 1.0000x reference)
import jax
import jax.numpy as jnp
from jax.experimental import pallas as pl


def kernel(x, edge_index, W1a, b1a, W1b, b1b, eps1, W2a, b2a, W2b, b2b, eps2):
    raise NotImplementedError("write your pallas kernel here")


# speedup vs baseline: 5.4089x; 5.4089x over previous
import functools

import jax
import jax.numpy as jnp
from jax import lax
from jax.experimental import pallas as pl
from jax.experimental.pallas import tpu as pltpu
from jax.experimental.pallas import tpu_sc as plsc

_NC = 2
_NS = 16
_K = 128


def _round_up(a, b):
    return (a + b - 1) // b * b


@functools.cache
def _make_seg_sum(n_acc, feat, cpw):
    mesh = plsc.VectorSubcoreMesh(
        core_axis_name="c", subcore_axis_name="s",
        num_cores=_NC, num_subcores=_NS)
    rt = n_acc // _NS

    @functools.partial(
        pl.kernel,
        out_type=jax.ShapeDtypeStruct((_NC, n_acc, feat), jnp.float32),
        mesh=mesh,
        scratch_types=[
            pltpu.VMEM((_K,), jnp.int32),
            pltpu.VMEM((_K,), jnp.int32),
            pltpu.VMEM((_K, feat), jnp.float32),
            pltpu.VMEM_SHARED((n_acc, feat), jnp.float32),
            pltpu.SemaphoreType.DMA,
        ],
        compiler_params=pltpu.CompilerParams(use_tc_tiling_on_sc=False),
    )
    def seg_sum(feat_hbm, src_hbm, dst_hbm, zero_hbm, out_hbm,
                src_v, dst_v, rows_v, acc_sh, sem):
        c = lax.axis_index("c")
        s = lax.axis_index("s")
        wid = s * _NC + c
        row0 = s * rt
        pltpu.sync_copy(zero_hbm.at[pl.ds(row0, rt)],
                        acc_sh.at[pl.ds(row0, rt)])
        plsc.subcore_barrier()

        def body(j, carry):
            base = (wid * cpw + j) * _K
            pltpu.sync_copy(src_hbm.at[pl.ds(base, _K)], src_v)
            pltpu.sync_copy(dst_hbm.at[pl.ds(base, _K)], dst_v)
            pltpu.async_copy(feat_hbm.at[src_v], rows_v, sem).wait()
            pltpu.sync_copy(rows_v, acc_sh.at[dst_v], add=True)
            return carry

        lax.fori_loop(0, cpw, body, 0)
        plsc.subcore_barrier()
        pltpu.sync_copy(acc_sh.at[pl.ds(row0, rt)],
                        out_hbm.at[c, pl.ds(row0, rt)])

    return seg_sum


def _mm_body(x_ref, w_ref, o_ref):
    o_ref[...] = jnp.dot(x_ref[...], w_ref[...],
                         preferred_element_type=jnp.float32)


def _mlp2_body(scale_ref, pre_ref, a0_ref, a1_ref, ba_ref, wb_ref, bb_ref,
               w2_ref, emb_ref, pre2_ref):
    t = (scale_ref[...] * pre_ref[...] + a0_ref[...] + a1_ref[...]
         + ba_ref[...])
    e = jnp.dot(jnp.maximum(t, 0.0), wb_ref[...],
                preferred_element_type=jnp.float32) + bb_ref[...]
    emb_ref[...] = e
    pre2_ref[...] = jnp.dot(jnp.maximum(e, 0.0), w2_ref[...],
                            preferred_element_type=jnp.float32)


def _mlp1_body(scale_ref, pre_ref, a0_ref, a1_ref, ba_ref, wb_ref, bb_ref,
               o_ref):
    t = (scale_ref[...] * pre_ref[...] + a0_ref[...] + a1_ref[...]
         + ba_ref[...])
    o_ref[...] = jnp.dot(jnp.maximum(t, 0.0), wb_ref[...],
                         preferred_element_type=jnp.float32) + bb_ref[...]


def kernel(x, edge_index, W1a, b1a, W1b, b1b, eps1, W2a, b2a, W2b, b2b, eps2):
    n, nf = x.shape
    hc = W1a.shape[1]
    nc = W2b.shape[1]
    e = edge_index.shape[1]

    cpw = -(-e // (_NC * _NS * _K))
    e_pad = _NC * _NS * _K * cpw
    src = edge_index[0]
    dst = edge_index[1]
    if e_pad > e:
        pad = e_pad - e
        src = jnp.concatenate([src, jnp.zeros((pad,), jnp.int32)])
        dst = jnp.concatenate([dst, jnp.full((pad,), n, jnp.int32)])

    n_acc = _round_up(n + 1, _NS * 8)
    zeros_acc = jnp.zeros((n_acc, hc), jnp.float32)
    seg_sum = _make_seg_sum(n_acc, hc, cpw)

    bk = 1000
    grid = (n // bk,)
    row_spec = lambda w: pl.BlockSpec((bk, w), lambda i: (i, 0))
    bcast_spec = lambda w: pl.BlockSpec((1, w), lambda i: (0, 0))
    sq_spec = lambda w: pl.BlockSpec((w, w), lambda i: (0, 0))

    pre1 = pl.pallas_call(
        _mm_body,
        grid=grid,
        in_specs=[row_spec(nf), pl.BlockSpec((nf, hc), lambda i: (0, 0))],
        out_specs=row_spec(hc),
        out_shape=jax.ShapeDtypeStruct((n, hc), jnp.float32),
    )(x, W1a)

    aggA = seg_sum(pre1, src, dst, zeros_acc)

    scale1 = jnp.full((1, hc), 1.0, jnp.float32) + eps1
    emb, pre2 = pl.pallas_call(
        _mlp2_body,
        grid=grid,
        in_specs=[bcast_spec(hc), row_spec(hc), row_spec(hc), row_spec(hc),
                  bcast_spec(hc), sq_spec(hc), bcast_spec(hc), sq_spec(hc)],
        out_specs=[row_spec(hc), row_spec(hc)],
        out_shape=[jax.ShapeDtypeStruct((n, hc), jnp.float32),
                   jax.ShapeDtypeStruct((n, hc), jnp.float32)],
    )(scale1, pre1, aggA[0, :n], aggA[1, :n], b1a.reshape(1, hc), W1b,
      b1b.reshape(1, hc), W2a)

    aggB = seg_sum(pre2, src, dst, zeros_acc)

    scale2 = jnp.full((1, hc), 1.0, jnp.float32) + eps2
    logits = pl.pallas_call(
        _mlp1_body,
        grid=grid,
        in_specs=[bcast_spec(hc), row_spec(hc), row_spec(hc), row_spec(hc),
                  bcast_spec(hc), pl.BlockSpec((hc, nc), lambda i: (0, 0)),
                  bcast_spec(nc)],
        out_specs=row_spec(nc),
        out_shape=jax.ShapeDtypeStruct((n, nc), jnp.float32),
    )(scale2, pre2, aggB[0, :n], aggB[1, :n], b2a.reshape(1, hc), W2b,
      b2b.reshape(1, nc))

    return (logits, emb)
